# baseline (speedup 1.0000x reference)
import functools

import jax
import jax.numpy as jnp
from jax import lax
from jax.experimental import pallas as pl
from jax.experimental.pallas import tpu as pltpu
from jax.experimental.pallas import tpu_sc as plsc


_SC_CHUNK = 16


def _sc_info():
    info = plsc.get_sparse_core_info()
    return info.num_cores, info.num_subcores


def _sc_gather_rows(table, idx):
    b = idx.shape[0]
    d = table.shape[1]
    nc, ns = _sc_info()
    nw = nc * ns
    bpw = b // nw
    ch = min(_SC_CHUNK, bpw)
    nch = bpw // ch
    mesh = plsc.VectorSubcoreMesh(core_axis_name="c", subcore_axis_name="s")

    @functools.partial(
        pl.kernel, mesh=mesh,
        out_type=jax.ShapeDtypeStruct((b, d), table.dtype),
        scratch_types=[
            pltpu.VMEM((2, ch), jnp.int32),
            pltpu.VMEM((2, ch, d), table.dtype),
            pltpu.SemaphoreType.DMA,
            pltpu.SemaphoreType.DMA,
        ],
    )
    def k(table_hbm, idx_hbm, out_hbm, idx_v, rows_v, sem0, sem1):
        wid = lax.axis_index("s") * nc + lax.axis_index("c")
        base = wid * bpw
        sems = (sem0, sem1)
        pending = [None, None]
        pltpu.sync_copy(idx_hbm.at[pl.ds(base, ch)], idx_v.at[0])
        pending[0] = pltpu.async_copy(
            table_hbm.at[idx_v.at[0]], rows_v.at[0], sems[0])
        for c in range(nch):
            cur = c % 2
            if c + 1 < nch:
                nb = (c + 1) % 2
                off = base + (c + 1) * ch
                pltpu.sync_copy(idx_hbm.at[pl.ds(off, ch)], idx_v.at[nb])
                pending[nb] = pltpu.async_copy(
                    table_hbm.at[idx_v.at[nb]], rows_v.at[nb], sems[nb])
            pending[cur].wait()
            pltpu.sync_copy(rows_v.at[cur], out_hbm.at[pl.ds(base + c * ch, ch)])

    return k(table, idx)


def _sc_scatter_rows(rows, idx, out_rows):
    b, d = rows.shape
    nc, ns = _sc_info()
    nw = nc * ns
    bpw = b // nw
    ch = min(_SC_CHUNK, bpw)
    nch = bpw // ch
    mesh = plsc.VectorSubcoreMesh(core_axis_name="c", subcore_axis_name="s")

    @functools.partial(
        pl.kernel, mesh=mesh,
        out_type=jax.ShapeDtypeStruct((out_rows, d), rows.dtype),
        scratch_types=[
            pltpu.VMEM((2, ch), jnp.int32),
            pltpu.VMEM((2, ch, d), rows.dtype),
            pltpu.SemaphoreType.DMA,
            pltpu.SemaphoreType.DMA,
        ],
    )
    def k(rows_hbm, idx_hbm, out_hbm, idx_v, rows_v, sem0, sem1):
        wid = lax.axis_index("s") * nc + lax.axis_index("c")
        base = wid * bpw
        sems = (sem0, sem1)
        pending = [None, None]
        pltpu.sync_copy(idx_hbm.at[pl.ds(base, ch)], idx_v.at[0])
        pltpu.sync_copy(rows_hbm.at[pl.ds(base, ch)], rows_v.at[0])
        pending[0] = pltpu.async_copy(
            rows_v.at[0], out_hbm.at[idx_v.at[0]], sems[0])
        for c in range(nch):
            if c + 1 < nch:
                nb = (c + 1) % 2
                off = base + (c + 1) * ch
                if pending[nb] is not None:
                    pending[nb].wait()
                pltpu.sync_copy(idx_hbm.at[pl.ds(off, ch)], idx_v.at[nb])
                pltpu.sync_copy(rows_hbm.at[pl.ds(off, ch)], rows_v.at[nb])
                pending[nb] = pltpu.async_copy(
                    rows_v.at[nb], out_hbm.at[idx_v.at[nb]], sems[nb])
        pending[(nch - 1) % 2].wait()
        if nch > 1:
            pending[(nch - 2) % 2].wait()

    return k(rows, idx)


def _moe_gemm_body(meta_ref, xg_ref, w1_ref, w3_ref, w2_ref, out_ref):
    i = pl.program_id(0)
    j = pl.program_id(1)
    nt = pl.num_programs(0)
    n_active = meta_ref[nt]

    @pl.when(i < n_active)
    def _():
        x = xg_ref[...]
        w1b = w1_ref[0]
        w3b = w3_ref[0]
        w2b = w2_ref[0]
        g = jax.lax.dot_general(x, w1b, (((1,), (1,)), ((), ())),
                                preferred_element_type=jnp.float32)
        u = jax.lax.dot_general(x, w3b, (((1,), (1,)), ((), ())),
                                preferred_element_type=jnp.float32)
        h = (g * jax.nn.sigmoid(g)) * u
        y = jax.lax.dot_general(h, w2b, (((1,), (1,)), ((), ())),
                                preferred_element_type=jnp.float32)

        @pl.when(j == 0)
        def _():
            out_ref[...] = y

        @pl.when(j > 0)
        def _():
            out_ref[...] += y


def _grouped_gemm(meta, xg, w1, w3, w2, *, tile_t, tile_f):
    nt = xg.shape[0] // tile_t
    nf = w1.shape[1] // tile_f
    d = xg.shape[1]

    def _live(i, s):
        return i < s[nt]

    def _xg_map(i, j, s):
        return (jnp.where(_live(i, s), i, 0), 0)

    def _w13_map(i, j, s):
        live = _live(i, s)
        return (jnp.where(live, s[i], 0), jnp.where(live, j, 0), 0)

    def _w2_map(i, j, s):
        live = _live(i, s)
        return (jnp.where(live, s[i], 0), 0, jnp.where(live, j, 0))

    def _out_map(i, j, s):
        return (jnp.where(_live(i, s), i, s[nt]), 0)

    grid_spec = pltpu.PrefetchScalarGridSpec(
        num_scalar_prefetch=1,
        grid=(nt, nf),
        in_specs=[
            pl.BlockSpec((tile_t, d), _xg_map),
            pl.BlockSpec((1, tile_f, d), _w13_map),
            pl.BlockSpec((1, tile_f, d), _w13_map),
            pl.BlockSpec((1, d, tile_f), _w2_map),
        ],
        out_specs=pl.BlockSpec((tile_t, d), _out_map),
    )
    return pl.pallas_call(
        _moe_gemm_body,
        grid_spec=grid_spec,
        out_shape=jax.ShapeDtypeStruct(((nt + 1) * tile_t, d), jnp.float32),
        compiler_params=pltpu.CompilerParams(
            dimension_semantics=("arbitrary", "arbitrary"),
        ),
    )(meta, xg, w1, w3, w2)


def kernel(x, router_logits, w1, w3, w2):
    t, d = x.shape
    e, f, _ = w1.shape
    tile_t = min(576, t)
    tile_f = min(512, f)
    nt = (t + e * (tile_t - 1)) // tile_t
    pcap = nt * tile_t

    eid = jnp.argmax(router_logits, axis=-1)
    top = jnp.max(router_logits, axis=-1)
    scale = jax.nn.sigmoid(top)
    xs = x * scale[:, None]

    onehot = (eid[:, None] == jnp.arange(e)[None, :]).astype(jnp.int32)
    counts = jnp.sum(onehot, axis=0)
    pc = ((counts + tile_t - 1) // tile_t) * tile_t
    ends = jnp.cumsum(pc)
    starts = ends - pc
    rank = jnp.cumsum(onehot, axis=0) - 1
    rank_t = jnp.take_along_axis(rank, eid[:, None], axis=1)[:, 0]
    pos = (starts[eid] + rank_t).astype(jnp.int32)

    tile_start = jnp.arange(nt) * tile_t
    tile_eid = jnp.searchsorted(ends, tile_start, side="right")
    tile_eid = jnp.minimum(tile_eid, e - 1).astype(jnp.int32)
    n_active = (ends[-1] // tile_t).astype(jnp.int32)
    meta = jnp.concatenate([tile_eid, n_active[None]])

    xg = _sc_scatter_rows(xs, pos, pcap)

    y = _grouped_gemm(meta, xg, w1, w3, w2, tile_t=tile_t, tile_f=tile_f)

    return _sc_gather_rows(y, pos)


# speedup vs baseline: 1.0123x; 1.0123x over previous
import functools

import jax
import jax.numpy as jnp
from jax import lax
from jax.experimental import pallas as pl
from jax.experimental.pallas import tpu as pltpu
from jax.experimental.pallas import tpu_sc as plsc


_SC_CHUNK = 32


def _sc_info():
    info = plsc.get_sparse_core_info()
    return info.num_cores, info.num_subcores


def _sc_gather_rows(table, idx):
    b = idx.shape[0]
    d = table.shape[1]
    nc, ns = _sc_info()
    nw = nc * ns
    bpw = b // nw
    ch = min(_SC_CHUNK, bpw)
    nch = bpw // ch
    mesh = plsc.VectorSubcoreMesh(core_axis_name="c", subcore_axis_name="s")

    @functools.partial(
        pl.kernel, mesh=mesh,
        out_type=jax.ShapeDtypeStruct((b, d), table.dtype),
        scratch_types=[
            pltpu.VMEM((ch,), jnp.int32),
            pltpu.VMEM((ch, d), table.dtype),
            pltpu.SemaphoreType.DMA,
        ],
    )
    def k(table_hbm, idx_hbm, out_hbm, idx_v, rows_v, sem):
        wid = lax.axis_index("s") * nc + lax.axis_index("c")
        base = wid * bpw
        for c in range(nch):
            off = base + c * ch
            pltpu.sync_copy(idx_hbm.at[pl.ds(off, ch)], idx_v)
            pltpu.async_copy(table_hbm.at[idx_v], rows_v, sem).wait()
            pltpu.sync_copy(rows_v, out_hbm.at[pl.ds(off, ch)])

    return k(table, idx)


def _sc_scatter_rows(rows, idx, out_rows):
    b, d = rows.shape
    nc, ns = _sc_info()
    nw = nc * ns
    bpw = b // nw
    ch = min(_SC_CHUNK, bpw)
    nch = bpw // ch
    mesh = plsc.VectorSubcoreMesh(core_axis_name="c", subcore_axis_name="s")

    @functools.partial(
        pl.kernel, mesh=mesh,
        out_type=jax.ShapeDtypeStruct((out_rows, d), rows.dtype),
        scratch_types=[
            pltpu.VMEM((ch,), jnp.int32),
            pltpu.VMEM((ch, d), rows.dtype),
            pltpu.SemaphoreType.DMA,
        ],
    )
    def k(rows_hbm, idx_hbm, out_hbm, idx_v, rows_v, sem):
        wid = lax.axis_index("s") * nc + lax.axis_index("c")
        base = wid * bpw
        for c in range(nch):
            off = base + c * ch
            pltpu.sync_copy(idx_hbm.at[pl.ds(off, ch)], idx_v)
            pltpu.sync_copy(rows_hbm.at[pl.ds(off, ch)], rows_v)
            pltpu.async_copy(rows_v, out_hbm.at[idx_v], sem).wait()

    return k(rows, idx)


def _moe_gemm_body(meta_ref, xg_ref, w1_ref, w3_ref, w2_ref, out_ref):
    i = pl.program_id(0)
    j = pl.program_id(1)
    nt = pl.num_programs(0)
    n_active = meta_ref[nt]

    @pl.when(i < n_active)
    def _():
        x = xg_ref[...]
        w1b = w1_ref[0]
        w3b = w3_ref[0]
        w2b = w2_ref[0]
        g = jax.lax.dot_general(x, w1b, (((1,), (1,)), ((), ())),
                                preferred_element_type=jnp.float32)
        u = jax.lax.dot_general(x, w3b, (((1,), (1,)), ((), ())),
                                preferred_element_type=jnp.float32)
        h = (g * jax.nn.sigmoid(g)) * u
        y = jax.lax.dot_general(h, w2b, (((1,), (1,)), ((), ())),
                                preferred_element_type=jnp.float32)

        @pl.when(j == 0)
        def _():
            out_ref[...] = y

        @pl.when(j > 0)
        def _():
            out_ref[...] += y


def _grouped_gemm(meta, xg, w1, w3, w2, *, tile_t, tile_f):
    nt = xg.shape[0] // tile_t
    nf = w1.shape[1] // tile_f
    d = xg.shape[1]

    def _live(i, s):
        return i < s[nt]

    def _xg_map(i, j, s):
        return (jnp.where(_live(i, s), i, 0), 0)

    def _w13_map(i, j, s):
        live = _live(i, s)
        return (jnp.where(live, s[i], 0), jnp.where(live, j, 0), 0)

    def _w2_map(i, j, s):
        live = _live(i, s)
        return (jnp.where(live, s[i], 0), 0, jnp.where(live, j, 0))

    def _out_map(i, j, s):
        return (jnp.where(_live(i, s), i, s[nt]), 0)

    grid_spec = pltpu.PrefetchScalarGridSpec(
        num_scalar_prefetch=1,
        grid=(nt, nf),
        in_specs=[
            pl.BlockSpec((tile_t, d), _xg_map),
            pl.BlockSpec((1, tile_f, d), _w13_map),
            pl.BlockSpec((1, tile_f, d), _w13_map),
            pl.BlockSpec((1, d, tile_f), _w2_map),
        ],
        out_specs=pl.BlockSpec((tile_t, d), _out_map),
    )
    return pl.pallas_call(
        _moe_gemm_body,
        grid_spec=grid_spec,
        out_shape=jax.ShapeDtypeStruct(((nt + 1) * tile_t, d), jnp.float32),
        compiler_params=pltpu.CompilerParams(
            dimension_semantics=("arbitrary", "arbitrary"),
        ),
    )(meta, xg, w1, w3, w2)


def kernel(x, router_logits, w1, w3, w2):
    t, d = x.shape
    e, f, _ = w1.shape
    tile_t = min(576, t)
    tile_f = min(512, f)
    nt = (t + e * (tile_t - 1)) // tile_t
    pcap = nt * tile_t

    eid = jnp.argmax(router_logits, axis=-1)
    top = jnp.max(router_logits, axis=-1)
    scale = jax.nn.sigmoid(top)
    xs = x * scale[:, None]

    onehot = (eid[:, None] == jnp.arange(e)[None, :]).astype(jnp.int32)
    counts = jnp.sum(onehot, axis=0)
    pc = ((counts + tile_t - 1) // tile_t) * tile_t
    ends = jnp.cumsum(pc)
    starts = ends - pc
    rank = jnp.cumsum(onehot, axis=0) - 1
    rank_t = jnp.sum(rank * onehot, axis=1)
    pos = (starts[eid] + rank_t).astype(jnp.int32)

    tile_start = jnp.arange(nt) * tile_t
    tile_eid = jnp.searchsorted(ends, tile_start, side="right")
    tile_eid = jnp.minimum(tile_eid, e - 1).astype(jnp.int32)
    n_active = (ends[-1] // tile_t).astype(jnp.int32)
    meta = jnp.concatenate([tile_eid, n_active[None]])

    xg = _sc_scatter_rows(xs, pos, pcap)

    y = _grouped_gemm(meta, xg, w1, w3, w2, tile_t=tile_t, tile_f=tile_f)

    return _sc_gather_rows(y, pos)
